# baseline (speedup 1.0000x reference)
import functools

import jax
import jax.numpy as jnp
from jax import lax
from jax.experimental import pallas as pl
from jax.experimental.pallas import tpu as pltpu
from jax.experimental.pallas import tpu_sc as plsc

VOCAB_SIZE = 100000
HIDDEN = 128
BATCH = 1024
SEQ_LEN = 200
NTOK = BATCH * SEQ_LEN

NUM_CORES = 2
NUM_SUBCORES = 16
NW = NUM_CORES * NUM_SUBCORES
TOK_PER_W = NTOK // NW
CHUNK = 80
STEPS = TOK_PER_W // CHUNK

_MESH = plsc.VectorSubcoreMesh(core_axis_name="c", subcore_axis_name="s")


NBUF = 8
GPB = 1
BLOCK = GPB * CHUNK
NBLK = TOK_PER_W // BLOCK


@functools.partial(
    pl.kernel,
    out_type=jax.ShapeDtypeStruct((NTOK, HIDDEN), jnp.float32),
    mesh=_MESH,
    scratch_types=[
        pltpu.VMEM((STEPS, CHUNK), jnp.int32),
        pltpu.VMEM((NBUF, BLOCK, HIDDEN), jnp.float32),
        [pltpu.SemaphoreType.DMA] * NBUF,
        [pltpu.SemaphoreType.DMA] * NBUF,
    ],
)
def _embed_sc(idx_hbm, table_hbm, out_hbm, idx_v, rows_v, gsems, osems):
    wid = lax.axis_index("s") * NUM_CORES + lax.axis_index("c")
    base = wid * TOK_PER_W
    pltpu.sync_copy(idx_hbm.at[wid], idx_v)

    def start_gathers(k, b):
        for g in range(GPB):
            pltpu.async_copy(
                table_hbm.at[idx_v.at[GPB * k + g]],
                rows_v.at[b, pl.ds(g * CHUNK, CHUNK)],
                gsems[b],
            )

    def wait_gathers(k, b):
        for g in range(GPB):
            pltpu.make_async_copy(
                table_hbm.at[idx_v.at[GPB * k + g]],
                rows_v.at[b, pl.ds(g * CHUNK, CHUNK)],
                gsems[b],
            ).wait()

    def start_out(k, b):
        pltpu.async_copy(
            rows_v.at[b], out_hbm.at[pl.ds(base + k * BLOCK, BLOCK)], osems[b]
        )

    def wait_out(k, b):
        pltpu.make_async_copy(
            rows_v.at[b], out_hbm.at[pl.ds(base + k * BLOCK, BLOCK)], osems[b]
        ).wait()

    for b in range(NBUF):
        start_gathers(b, b)

    @pl.loop(0, NBLK - NBUF, step=NBUF)
    def _steady(k0):
        for b in range(NBUF):
            k = k0 + b
            wait_gathers(k, b)
            start_out(k, b)
            wait_out(k, b)
            start_gathers(k + NBUF, b)

    for b in range(NBUF):
        k = NBLK - NBUF + b
        wait_gathers(k, b)
        start_out(k, b)
        wait_out(k, b)


def kernel(input_ids, weight):
    idx = input_ids.reshape(NW, STEPS, CHUNK).astype(jnp.int32)
    out = _embed_sc(idx, weight)
    return out.reshape(BATCH, SEQ_LEN, HIDDEN)


# speedup vs baseline: 1.0017x; 1.0017x over previous
import functools

import jax
import jax.numpy as jnp
from jax import lax
from jax.experimental import pallas as pl
from jax.experimental.pallas import tpu as pltpu
from jax.experimental.pallas import tpu_sc as plsc

VOCAB_SIZE = 100000
HIDDEN = 128
BATCH = 1024
SEQ_LEN = 200
NTOK = BATCH * SEQ_LEN

NUM_CORES = 2
NUM_SUBCORES = 16
NW = NUM_CORES * NUM_SUBCORES
TOK_PER_W = NTOK // NW
CHUNK = 80
NBLK = TOK_PER_W // CHUNK
NBUF = 8

_MESH = plsc.VectorSubcoreMesh(core_axis_name="c", subcore_axis_name="s")


@functools.partial(
    pl.kernel,
    out_type=jax.ShapeDtypeStruct((NTOK, HIDDEN), jnp.float32),
    mesh=_MESH,
    scratch_types=[
        pltpu.VMEM((NBLK, CHUNK), jnp.int32),
        pltpu.VMEM((NBUF, CHUNK, HIDDEN), jnp.float32),
        [pltpu.SemaphoreType.DMA] * NBUF,
        [pltpu.SemaphoreType.DMA] * NBUF,
    ],
)
def _embed_sc(idx_hbm, table_hbm, out_hbm, idx_v, rows_v, gsems, osems):
    wid = lax.axis_index("s") * NUM_CORES + lax.axis_index("c")
    base = wid * TOK_PER_W
    pltpu.sync_copy(idx_hbm.at[wid], idx_v)

    def start_gather(k, b):
        pltpu.async_copy(table_hbm.at[idx_v.at[k]], rows_v.at[b], gsems[b])

    def wait_gather(k, b):
        pltpu.make_async_copy(table_hbm.at[idx_v.at[k]], rows_v.at[b], gsems[b]).wait()

    def start_out(k, b):
        pltpu.async_copy(
            rows_v.at[b], out_hbm.at[pl.ds(base + k * CHUNK, CHUNK)], osems[b]
        )

    def wait_out(k, b):
        pltpu.make_async_copy(
            rows_v.at[b], out_hbm.at[pl.ds(base + k * CHUNK, CHUNK)], osems[b]
        ).wait()

    for b in range(NBUF):
        start_gather(b, b)

    @pl.loop(0, NBLK - NBUF, step=NBUF)
    def _steady(k0):
        for b in range(NBUF):
            k = k0 + b
            wait_gather(k, b)
            start_out(k, b)
            wait_out(k, b)
            start_gather(k + NBUF, b)

    for b in range(NBUF):
        k = NBLK - NBUF + b
        wait_gather(k, b)
        start_out(k, b)
        wait_out(k, b)


def kernel(input_ids, weight):
    idx = input_ids.reshape(NW, NBLK, CHUNK).astype(jnp.int32)
    out = _embed_sc(idx, weight)
    return out.reshape(BATCH, SEQ_LEN, HIDDEN)
